# baseline (device time: 327698 ns/iter reference)
import jax
import jax.numpy as jnp
from jax import lax
from jax.experimental import pallas as pl
from jax.experimental.pallas import tpu as pltpu

N_DEV = 4


def kernel(A, B):
    m, k = A.shape
    k2, n = B.shape

    def body(a_ref, b_ref, out_ref, comm_ref, send_sems, recv_sems):
        my = lax.axis_index("i")
        left = (my - 1) % N_DEV
        right = (my + 1) % N_DEV

        barrier_sem = pltpu.get_barrier_semaphore()
        for nbr in [left, right]:
            pl.semaphore_signal(
                barrier_sem, inc=1,
                device_id=(nbr,), device_id_type=pl.DeviceIdType.MESH,
            )
        pl.semaphore_wait(barrier_sem, 2)

        partial = jnp.dot(a_ref[...], b_ref[...],
                          preferred_element_type=jnp.float32)
        out_ref[...] = partial
        comm_ref[0, :, :] = partial

        for h in range(N_DEV - 1):
            send_slot = h % 2
            recv_slot = (h + 1) % 2
            rdma = pltpu.make_async_remote_copy(
                src_ref=comm_ref.at[send_slot],
                dst_ref=comm_ref.at[recv_slot],
                send_sem=send_sems.at[send_slot],
                recv_sem=recv_sems.at[recv_slot],
                device_id=(right,),
                device_id_type=pl.DeviceIdType.MESH,
            )
            rdma.start()
            rdma.wait()
            out_ref[...] += comm_ref[recv_slot, :, :]

        out_ref[...] = jnp.maximum(out_ref[...], 0.0)

    return pl.pallas_call(
        body,
        out_shape=jax.ShapeDtypeStruct((m, n), jnp.float32),
        in_specs=[
            pl.BlockSpec(memory_space=pltpu.VMEM),
            pl.BlockSpec(memory_space=pltpu.VMEM),
        ],
        out_specs=pl.BlockSpec(memory_space=pltpu.VMEM),
        scratch_shapes=[
            pltpu.VMEM((2, m, n), jnp.float32),
            pltpu.SemaphoreType.DMA((2,)),
            pltpu.SemaphoreType.DMA((2,)),
        ],
        compiler_params=pltpu.CompilerParams(collective_id=0),
    )(A, B)


# device time: 174854 ns/iter; 1.8741x vs baseline; 1.8741x over previous
import jax
import jax.numpy as jnp
from jax import lax
from jax.experimental import pallas as pl
from jax.experimental.pallas import tpu as pltpu

N_DEV = 4


def kernel(A, B):
    m, k = A.shape
    k2, n = B.shape
    C = m // N_DEV

    def body(a_ref, b_ref, out_ref, send_buf, recv_buf,
             rs_send_sems, rs_recv_sems, ag_send_sems, ag_recv_sems):
        d = lax.axis_index("i")
        left = (d + N_DEV - 1) % N_DEV
        right = (d + 1) % N_DEV

        barrier_sem = pltpu.get_barrier_semaphore()
        for nbr in (left, right):
            pl.semaphore_signal(
                barrier_sem, inc=1,
                device_id=(nbr,), device_id_type=pl.DeviceIdType.MESH,
            )
        pl.semaphore_wait(barrier_sem, 2)

        def partial(c):
            return jnp.dot(a_ref[pl.ds(c * C, C), :], b_ref[...],
                           preferred_element_type=jnp.float32)

        c0 = (d + N_DEV - 1) % N_DEV
        send_buf[0, :, :] = partial(c0)

        rs_rdmas = []
        for s in range(N_DEV - 1):
            rdma = pltpu.make_async_remote_copy(
                src_ref=send_buf.at[s],
                dst_ref=recv_buf.at[s],
                send_sem=rs_send_sems.at[s],
                recv_sem=rs_recv_sems.at[s],
                device_id=(right,),
                device_id_type=pl.DeviceIdType.MESH,
            )
            rdma.start()
            rs_rdmas.append(rdma)
            c_next = (d + 2 * N_DEV - 2 - s) % N_DEV
            if s < N_DEV - 2:
                send_buf[s + 1, :, :] = partial(c_next)
                rdma.wait_recv()
                send_buf[s + 1, :, :] += recv_buf[s]
            else:
                out_ref[pl.ds(d * C, C), :] = partial(c_next)
                rdma.wait_recv()
                out_ref[pl.ds(d * C, C), :] = jnp.maximum(
                    out_ref[pl.ds(d * C, C), :] + recv_buf[s], 0.0)
        for rdma in rs_rdmas:
            rdma.wait_send()

        for h in range(N_DEV - 1):
            o = (d + N_DEV - h) % N_DEV
            rdma = pltpu.make_async_remote_copy(
                src_ref=out_ref.at[pl.ds(o * C, C)],
                dst_ref=out_ref.at[pl.ds(o * C, C)],
                send_sem=ag_send_sems.at[h],
                recv_sem=ag_recv_sems.at[h],
                device_id=(right,),
                device_id_type=pl.DeviceIdType.MESH,
            )
            rdma.start()
            rdma.wait()

    return pl.pallas_call(
        body,
        out_shape=jax.ShapeDtypeStruct((m, n), jnp.float32),
        in_specs=[
            pl.BlockSpec(memory_space=pltpu.VMEM),
            pl.BlockSpec(memory_space=pltpu.VMEM),
        ],
        out_specs=pl.BlockSpec(memory_space=pltpu.VMEM),
        scratch_shapes=[
            pltpu.VMEM((N_DEV - 1, C, n), jnp.float32),
            pltpu.VMEM((N_DEV - 1, C, n), jnp.float32),
            pltpu.SemaphoreType.DMA((N_DEV - 1,)),
            pltpu.SemaphoreType.DMA((N_DEV - 1,)),
            pltpu.SemaphoreType.DMA((N_DEV - 1,)),
            pltpu.SemaphoreType.DMA((N_DEV - 1,)),
        ],
        compiler_params=pltpu.CompilerParams(collective_id=0),
    )(A, B)


# device time: 65014 ns/iter; 5.0404x vs baseline; 2.6895x over previous
import jax
import jax.numpy as jnp
from jax import lax
from jax.experimental import pallas as pl
from jax.experimental.pallas import tpu as pltpu

N_DEV = 4


def kernel(A, B):
    m, k = A.shape
    k2, n = B.shape
    C = m // N_DEV
    H = n // 2

    def body(a_ref, b_ref, out_ref,
             cw_send, cw_recv, ccw_send, ccw_recv, ag_cw, ag_ccw,
             cw_ssem, cw_rsem, ccw_ssem, ccw_rsem,
             agcw_ssem, agcw_rsem, agccw_ssem, agccw_rsem):
        d = lax.axis_index("i")
        left = (d + N_DEV - 1) % N_DEV
        right = (d + 1) % N_DEV

        barrier_sem = pltpu.get_barrier_semaphore()
        for nbr in (left, right):
            pl.semaphore_signal(
                barrier_sem, inc=1,
                device_id=(nbr,), device_id_type=pl.DeviceIdType.MESH,
            )
        pl.semaphore_wait(barrier_sem, 2)

        f32 = jnp.float32

        def p_left(c):
            return jnp.dot(a_ref[pl.ds(c * C, C), :], b_ref[:, :H],
                           preferred_element_type=f32)

        def p_right(c):
            return jnp.dot(a_ref[pl.ds(c * C, C), :], b_ref[:, H:],
                           preferred_element_type=f32)

        cw_send[0, :, :] = p_left((d + N_DEV - 1) % N_DEV).astype(jnp.bfloat16)
        ccw_send[0, :, :] = p_right((d + 1) % N_DEV).astype(jnp.bfloat16)

        rs_rdmas = []
        for s in range(N_DEV - 1):
            cw = pltpu.make_async_remote_copy(
                src_ref=cw_send.at[s], dst_ref=cw_recv.at[s],
                send_sem=cw_ssem.at[s], recv_sem=cw_rsem.at[s],
                device_id=(right,), device_id_type=pl.DeviceIdType.MESH,
            )
            ccw = pltpu.make_async_remote_copy(
                src_ref=ccw_send.at[s], dst_ref=ccw_recv.at[s],
                send_sem=ccw_ssem.at[s], recv_sem=ccw_rsem.at[s],
                device_id=(left,), device_id_type=pl.DeviceIdType.MESH,
            )
            cw.start()
            ccw.start()
            rs_rdmas += [cw, ccw]
            if s < N_DEV - 2:
                c_cw = (d + 2 * N_DEV - 2 - s) % N_DEV
                c_ccw = (d + 2 + s) % N_DEV
                pcw = p_left(c_cw)
                pccw = p_right(c_ccw)
                cw.wait_recv()
                cw_send[s + 1, :, :] = (
                    pcw + cw_recv[s].astype(f32)).astype(jnp.bfloat16)
                ccw.wait_recv()
                ccw_send[s + 1, :, :] = (
                    pccw + ccw_recv[s].astype(f32)).astype(jnp.bfloat16)
            else:
                rows = pl.ds(d * C, C)
                pfull = jnp.dot(a_ref[rows, :], b_ref[...],
                                preferred_element_type=f32)
                cw.wait_recv()
                ccw.wait_recv()
                lh = jnp.maximum(pfull[:, :H] + cw_recv[s].astype(f32), 0.0)
                rh = jnp.maximum(pfull[:, H:] + ccw_recv[s].astype(f32), 0.0)
                out_ref[rows, :H] = lh
                out_ref[rows, H:] = rh
                ag_cw[0, :, :] = lh.astype(jnp.bfloat16)
                ag_ccw[0, :, :] = rh.astype(jnp.bfloat16)
        for r in rs_rdmas:
            r.wait_send()

        ag_rdmas = []
        for h in range(N_DEV - 1):
            cw = pltpu.make_async_remote_copy(
                src_ref=ag_cw.at[h], dst_ref=ag_cw.at[h + 1],
                send_sem=agcw_ssem.at[h], recv_sem=agcw_rsem.at[h],
                device_id=(right,), device_id_type=pl.DeviceIdType.MESH,
            )
            ccw = pltpu.make_async_remote_copy(
                src_ref=ag_ccw.at[h], dst_ref=ag_ccw.at[h + 1],
                send_sem=agccw_ssem.at[h], recv_sem=agccw_rsem.at[h],
                device_id=(left,), device_id_type=pl.DeviceIdType.MESH,
            )
            cw.start()
            ccw.start()
            ag_rdmas += [cw, ccw]
            cw.wait_recv()
            ccw.wait_recv()
            o_cw = (d + N_DEV - 1 - h) % N_DEV
            o_ccw = (d + 1 + h) % N_DEV
            out_ref[pl.ds(o_cw * C, C), :H] = ag_cw[h + 1].astype(f32)
            out_ref[pl.ds(o_ccw * C, C), H:] = ag_ccw[h + 1].astype(f32)
        for r in ag_rdmas:
            r.wait_send()

    return pl.pallas_call(
        body,
        out_shape=jax.ShapeDtypeStruct((m, n), jnp.float32),
        in_specs=[
            pl.BlockSpec(memory_space=pltpu.VMEM),
            pl.BlockSpec(memory_space=pltpu.VMEM),
        ],
        out_specs=pl.BlockSpec(memory_space=pltpu.VMEM),
        scratch_shapes=[
            pltpu.VMEM((N_DEV - 1, C, H), jnp.bfloat16),
            pltpu.VMEM((N_DEV - 1, C, H), jnp.bfloat16),
            pltpu.VMEM((N_DEV - 1, C, H), jnp.bfloat16),
            pltpu.VMEM((N_DEV - 1, C, H), jnp.bfloat16),
            pltpu.VMEM((N_DEV, C, H), jnp.bfloat16),
            pltpu.VMEM((N_DEV, C, H), jnp.bfloat16),
            pltpu.SemaphoreType.DMA((N_DEV - 1,)),
            pltpu.SemaphoreType.DMA((N_DEV - 1,)),
            pltpu.SemaphoreType.DMA((N_DEV - 1,)),
            pltpu.SemaphoreType.DMA((N_DEV - 1,)),
            pltpu.SemaphoreType.DMA((N_DEV - 1,)),
            pltpu.SemaphoreType.DMA((N_DEV - 1,)),
            pltpu.SemaphoreType.DMA((N_DEV - 1,)),
            pltpu.SemaphoreType.DMA((N_DEV - 1,)),
        ],
        compiler_params=pltpu.CompilerParams(collective_id=0),
    )(A, B)


# device time: 63229 ns/iter; 5.1827x vs baseline; 1.0282x over previous
import jax
import jax.numpy as jnp
from jax import lax
from jax.experimental import pallas as pl
from jax.experimental.pallas import tpu as pltpu

N_DEV = 4


def kernel(A, B):
    m, k = A.shape
    k2, n = B.shape
    C = m // N_DEV
    H = n // 2

    def body(a_ref, b_ref, out_ref, a16, b16,
             cw_send, cw_recv, ccw_send, ccw_recv, ag_cw, ag_ccw,
             cw_ssem, cw_rsem, ccw_ssem, ccw_rsem,
             agcw_ssem, agcw_rsem, agccw_ssem, agccw_rsem):
        d = lax.axis_index("i")
        left = (d + N_DEV - 1) % N_DEV
        right = (d + 1) % N_DEV

        barrier_sem = pltpu.get_barrier_semaphore()
        for nbr in (left, right):
            pl.semaphore_signal(
                barrier_sem, inc=1,
                device_id=(nbr,), device_id_type=pl.DeviceIdType.MESH,
            )
        pl.semaphore_wait(barrier_sem, 2)

        f32 = jnp.float32
        bf16 = jnp.bfloat16

        a16[...] = a_ref[...].astype(bf16)
        b16[...] = b_ref[...].astype(bf16)

        def p_left(c):
            return jnp.dot(a16[pl.ds(c * C, C), :], b16[:, :H],
                           preferred_element_type=f32)

        def p_right(c):
            return jnp.dot(a16[pl.ds(c * C, C), :], b16[:, H:],
                           preferred_element_type=f32)

        rs_cw, rs_ccw = [], []
        for s in range(N_DEV - 1):
            rs_cw.append(pltpu.make_async_remote_copy(
                src_ref=cw_send.at[s], dst_ref=cw_recv.at[s],
                send_sem=cw_ssem.at[s], recv_sem=cw_rsem.at[s],
                device_id=(right,), device_id_type=pl.DeviceIdType.MESH,
            ))
            rs_ccw.append(pltpu.make_async_remote_copy(
                src_ref=ccw_send.at[s], dst_ref=ccw_recv.at[s],
                send_sem=ccw_ssem.at[s], recv_sem=ccw_rsem.at[s],
                device_id=(left,), device_id_type=pl.DeviceIdType.MESH,
            ))
        ag_cws, ag_ccws = [], []
        for h in range(N_DEV - 1):
            ag_cws.append(pltpu.make_async_remote_copy(
                src_ref=ag_cw.at[h], dst_ref=ag_cw.at[h + 1],
                send_sem=agcw_ssem.at[h], recv_sem=agcw_rsem.at[h],
                device_id=(right,), device_id_type=pl.DeviceIdType.MESH,
            ))
            ag_ccws.append(pltpu.make_async_remote_copy(
                src_ref=ag_ccw.at[h], dst_ref=ag_ccw.at[h + 1],
                send_sem=agccw_ssem.at[h], recv_sem=agccw_rsem.at[h],
                device_id=(left,), device_id_type=pl.DeviceIdType.MESH,
            ))

        cw_send[0, :, :] = p_left((d + N_DEV - 1) % N_DEV).astype(bf16)
        rs_cw[0].start()
        ccw_send[0, :, :] = p_right((d + 1) % N_DEV).astype(bf16)
        rs_ccw[0].start()

        for s in range(N_DEV - 2):
            pcw = p_left((d + 2 * N_DEV - 2 - s) % N_DEV)
            rs_cw[s].wait_recv()
            cw_send[s + 1, :, :] = (pcw + cw_recv[s].astype(f32)).astype(bf16)
            rs_cw[s + 1].start()
            pccw = p_right((d + 2 + s) % N_DEV)
            rs_ccw[s].wait_recv()
            ccw_send[s + 1, :, :] = (
                pccw + ccw_recv[s].astype(f32)).astype(bf16)
            rs_ccw[s + 1].start()

        rows = pl.ds(d * C, C)
        pfull = jnp.dot(a16[rows, :], b16[...], preferred_element_type=f32)
        last = N_DEV - 2
        rs_cw[last].wait_recv()
        lh = jnp.maximum(pfull[:, :H] + cw_recv[last].astype(f32), 0.0)
        ag_cw[0, :, :] = lh.astype(bf16)
        ag_cws[0].start()
        rs_ccw[last].wait_recv()
        rh = jnp.maximum(pfull[:, H:] + ccw_recv[last].astype(f32), 0.0)
        ag_ccw[0, :, :] = rh.astype(bf16)
        ag_ccws[0].start()
        out_ref[rows, :H] = lh
        out_ref[rows, H:] = rh

        for h in range(N_DEV - 1):
            ag_cws[h].wait_recv()
            if h < N_DEV - 2:
                ag_cws[h + 1].start()
            ag_ccws[h].wait_recv()
            if h < N_DEV - 2:
                ag_ccws[h + 1].start()
            o_cw = (d + N_DEV - 1 - h) % N_DEV
            o_ccw = (d + 1 + h) % N_DEV
            out_ref[pl.ds(o_cw * C, C), :H] = ag_cw[h + 1].astype(f32)
            out_ref[pl.ds(o_ccw * C, C), H:] = ag_ccw[h + 1].astype(f32)

        for r in rs_cw + rs_ccw + ag_cws + ag_ccws:
            r.wait_send()

    return pl.pallas_call(
        body,
        out_shape=jax.ShapeDtypeStruct((m, n), jnp.float32),
        in_specs=[
            pl.BlockSpec(memory_space=pltpu.VMEM),
            pl.BlockSpec(memory_space=pltpu.VMEM),
        ],
        out_specs=pl.BlockSpec(memory_space=pltpu.VMEM),
        scratch_shapes=[
            pltpu.VMEM((m, k), jnp.bfloat16),
            pltpu.VMEM((k, n), jnp.bfloat16),
            pltpu.VMEM((N_DEV - 1, C, H), jnp.bfloat16),
            pltpu.VMEM((N_DEV - 1, C, H), jnp.bfloat16),
            pltpu.VMEM((N_DEV - 1, C, H), jnp.bfloat16),
            pltpu.VMEM((N_DEV - 1, C, H), jnp.bfloat16),
            pltpu.VMEM((N_DEV, C, H), jnp.bfloat16),
            pltpu.VMEM((N_DEV, C, H), jnp.bfloat16),
            pltpu.SemaphoreType.DMA((N_DEV - 1,)),
            pltpu.SemaphoreType.DMA((N_DEV - 1,)),
            pltpu.SemaphoreType.DMA((N_DEV - 1,)),
            pltpu.SemaphoreType.DMA((N_DEV - 1,)),
            pltpu.SemaphoreType.DMA((N_DEV - 1,)),
            pltpu.SemaphoreType.DMA((N_DEV - 1,)),
            pltpu.SemaphoreType.DMA((N_DEV - 1,)),
            pltpu.SemaphoreType.DMA((N_DEV - 1,)),
        ],
        compiler_params=pltpu.CompilerParams(collective_id=0),
    )(A, B)
